# baseline (speedup 1.0000x reference)
import functools

import jax
import jax.numpy as jnp
from jax import lax
from jax.experimental import pallas as pl
from jax.experimental.pallas import tpu as pltpu
from jax.experimental.pallas import tpu_sc as plsc

B = 1024
V = 100000
L = 16
NC, NS = 2, 16
NW = NC * NS
BPW = B // NW


@functools.partial(
    pl.kernel,
    mesh=plsc.VectorSubcoreMesh(core_axis_name="c", subcore_axis_name="s"),
    out_type=jax.ShapeDtypeStruct((B,), jnp.float32),
    scratch_types=[
        pltpu.VMEM((BPW,), jnp.int32),
        pltpu.VMEM((BPW,), jnp.int32),
        pltpu.VMEM((BPW,), jnp.float32),
        pltpu.VMEM((BPW,), jnp.float32),
        pltpu.VMEM((BPW,), jnp.float32),
        pltpu.SemaphoreType.DMA,
    ],
)
def _nll_sc(cost_flat, inputs_hbm, mask_hbm, out_hbm,
            idx_v, flat_v, vals_v, mask_v, out_v, sem):
    wid = lax.axis_index("s") * NC + lax.axis_index("c")
    base = wid * BPW

    pltpu.sync_copy(inputs_hbm.at[pl.ds(base, BPW)], idx_v)
    pltpu.sync_copy(mask_hbm.at[pl.ds(base, BPW)], mask_v)

    for j in range(BPW // L):
        row = base + j * L + lax.iota(jnp.int32, L)
        flat_v[pl.ds(j * L, L)] = idx_v[pl.ds(j * L, L)] + row * V

    pltpu.async_copy(cost_flat.at[flat_v], vals_v, sem).wait()

    for j in range(BPW // L):
        sl = pl.ds(j * L, L)
        out_v[sl] = -vals_v[sl] * mask_v[sl]

    pltpu.sync_copy(out_v, out_hbm.at[pl.ds(base, BPW)])


@jax.jit
def kernel(cost, inputs, mask):
    cost_flat = cost.reshape(B * V)
    return _nll_sc(cost_flat, inputs.astype(jnp.int32), mask)


# speedup vs baseline: 2.3534x; 2.3534x over previous
import functools

import jax
import jax.numpy as jnp
from jax import lax
from jax.experimental import pallas as pl
from jax.experimental.pallas import tpu as pltpu
from jax.experimental.pallas import tpu_sc as plsc

B = 1024
V = 100000
L = 16
NC, NS = 2, 16
NW = NC * NS
BPW = B // NW
TR, TC_ = 8, 128


@functools.partial(
    pl.kernel,
    mesh=plsc.VectorSubcoreMesh(core_axis_name="c", subcore_axis_name="s"),
    out_type=jax.ShapeDtypeStruct((B,), jnp.float32),
    compiler_params=pltpu.CompilerParams(needs_layout_passes=False),
    scratch_types=[
        pltpu.VMEM((BPW,), jnp.int32),
        pltpu.VMEM((BPW,), jnp.int32),
        pltpu.VMEM((BPW,), jnp.int32),
        pltpu.VMEM((BPW, TR, TC_), jnp.float32),
        pltpu.VMEM((BPW,), jnp.float32),
        pltpu.VMEM((BPW,), jnp.float32),
        pltpu.SemaphoreType.DMA,
    ],
)
def _nll_sc(cost_hbm, inputs_hbm, mask_hbm, out_hbm,
            idx_v, col_v, lane_v, tiles_v, mask_v, out_v, sem):
    wid = lax.axis_index("s") * NC + lax.axis_index("c")
    base = wid * BPW

    pltpu.sync_copy(inputs_hbm.at[pl.ds(base, BPW)], idx_v)
    pltpu.sync_copy(mask_hbm.at[pl.ds(base, BPW)], mask_v)

    for j in range(BPW // L):
        sl = pl.ds(j * L, L)
        col_v[sl] = lax.bitwise_and(idx_v[sl], ~(TC_ - 1))
        lane_v[sl] = lax.bitwise_and(idx_v[sl], TC_ - 1)

    cols = [col_v[pl.ds(j * L, L)] for j in range(BPW // L)]
    copies = []
    for j in range(BPW):
        col = pl.multiple_of(cols[j // L][j % L], TC_)
        row = pl.multiple_of(base + (j & ~(TR - 1)), TR)
        copies.append(pltpu.async_copy(
            cost_hbm.at[pl.ds(row, TR), pl.ds(col, TC_)],
            tiles_v.at[j], sem))
    for c in copies:
        c.wait()

    for j in range(BPW // L):
        sl = pl.ds(j * L, L)
        jidx = j * L + lax.iota(jnp.int32, L)
        subrow = lax.bitwise_and(jidx, TR - 1)
        vals = plsc.load_gather(tiles_v, [jidx, subrow, lane_v[sl]])
        out_v[sl] = -vals * mask_v[sl]

    pltpu.sync_copy(out_v, out_hbm.at[pl.ds(base, BPW)])


@jax.jit
def kernel(cost, inputs, mask):
    return _nll_sc(cost, inputs.astype(jnp.int32), mask)


# speedup vs baseline: 37.4855x; 15.9282x over previous
import functools

import jax
import jax.numpy as jnp
from jax import lax
from jax.experimental import pallas as pl
from jax.experimental.pallas import tpu as pltpu
from jax.experimental.pallas import tpu_sc as plsc

B = 1024
V = 100000
L = 16
NC, NS = 2, 16
NW = NC * NS
BPW = B // NW
TR, TL = 8, 128


@functools.partial(
    pl.kernel,
    mesh=plsc.VectorSubcoreMesh(core_axis_name="c", subcore_axis_name="s"),
    out_type=jax.ShapeDtypeStruct((B,), jnp.float32),
    compiler_params=pltpu.CompilerParams(needs_layout_passes=False),
    scratch_types=[
        pltpu.VMEM((BPW,), jnp.int32),
        pltpu.VMEM((BPW,), jnp.int32),
        pltpu.VMEM((BPW, TR, TL), jnp.float32),
        pltpu.VMEM((BPW,), jnp.float32),
        pltpu.VMEM((BPW,), jnp.float32),
        pltpu.SemaphoreType.DMA,
    ],
)
def _nll_sc(cost_t_hbm, inputs_hbm, mask_hbm, out_hbm,
            idx_v, row_v, tiles_v, mask_v, out_v, sem):
    wid = lax.axis_index("s") * NC + lax.axis_index("c")
    base = wid * BPW
    col_block = lax.bitwise_and(base, ~(TL - 1))
    col_off = lax.bitwise_and(base, TL - 1)

    pltpu.sync_copy(inputs_hbm.at[pl.ds(base, BPW)], idx_v)
    pltpu.sync_copy(mask_hbm.at[pl.ds(base, BPW)], mask_v)

    for j in range(BPW // L):
        sl = pl.ds(j * L, L)
        row_v[sl] = lax.bitwise_and(idx_v[sl], ~(TR - 1))

    rows = [row_v[pl.ds(j * L, L)] for j in range(BPW // L)]
    col = pl.multiple_of(col_block, TL)
    copies = []
    for j in range(BPW):
        row = pl.multiple_of(rows[j // L][j % L], TR)
        copies.append(pltpu.async_copy(
            cost_t_hbm.at[pl.ds(row, TR), pl.ds(col, TL)],
            tiles_v.at[j], sem))
    for c in copies:
        c.wait()

    for j in range(BPW // L):
        sl = pl.ds(j * L, L)
        jidx = j * L + lax.iota(jnp.int32, L)
        subrow = lax.bitwise_and(idx_v[sl], TR - 1)
        vals = plsc.load_gather(tiles_v, [jidx, subrow, col_off + jidx])
        out_v[sl] = -vals * mask_v[sl]

    pltpu.sync_copy(out_v, out_hbm.at[pl.ds(base, BPW)])


@jax.jit
def kernel(cost, inputs, mask):
    return _nll_sc(cost.T, inputs.astype(jnp.int32), mask)


# speedup vs baseline: 37.5932x; 1.0029x over previous
import functools

import jax
import jax.numpy as jnp
from jax import lax
from jax.experimental import pallas as pl
from jax.experimental.pallas import tpu as pltpu
from jax.experimental.pallas import tpu_sc as plsc

B = 1024
V = 100000
L = 16
NC, NS = 2, 16
NW = NC * NS
BPW = B // NW
TR, TL = 8, 128


@functools.partial(
    pl.kernel,
    mesh=plsc.VectorSubcoreMesh(core_axis_name="c", subcore_axis_name="s"),
    out_type=jax.ShapeDtypeStruct((B,), jnp.float32),
    compiler_params=pltpu.CompilerParams(
        needs_layout_passes=False, skip_device_barrier=True),
    scratch_types=[
        pltpu.VMEM((BPW,), jnp.int32),
        pltpu.VMEM((BPW,), jnp.int32),
        pltpu.VMEM((BPW, TR, TL), jnp.float32),
        pltpu.VMEM((BPW,), jnp.float32),
        pltpu.VMEM((BPW,), jnp.float32),
        pltpu.SemaphoreType.DMA,
    ],
)
def _nll_sc(cost_t_hbm, inputs_hbm, mask_hbm, out_hbm,
            idx_v, row_v, tiles_v, mask_v, out_v, sem):
    wid = lax.axis_index("s") * NC + lax.axis_index("c")
    base = wid * BPW
    col_block = lax.bitwise_and(base, ~(TL - 1))
    col_off = lax.bitwise_and(base, TL - 1)

    pltpu.sync_copy(inputs_hbm.at[pl.ds(base, BPW)], idx_v)
    pltpu.sync_copy(mask_hbm.at[pl.ds(base, BPW)], mask_v)

    for j in range(BPW // L):
        sl = pl.ds(j * L, L)
        row_v[sl] = lax.bitwise_and(idx_v[sl], ~(TR - 1))

    rows = [row_v[pl.ds(j * L, L)] for j in range(BPW // L)]
    col = pl.multiple_of(col_block, TL)
    copies = []
    for j in range(BPW):
        row = pl.multiple_of(rows[j // L][j % L], TR)
        copies.append(pltpu.async_copy(
            cost_t_hbm.at[pl.ds(row, TR), pl.ds(col, TL)],
            tiles_v.at[j], sem))
    for c in copies:
        c.wait()

    for j in range(BPW // L):
        sl = pl.ds(j * L, L)
        jidx = j * L + lax.iota(jnp.int32, L)
        subrow = lax.bitwise_and(idx_v[sl], TR - 1)
        vals = plsc.load_gather(tiles_v, [jidx, subrow, col_off + jidx])
        out_v[sl] = -vals * mask_v[sl]

    pltpu.sync_copy(out_v, out_hbm.at[pl.ds(base, BPW)])


@jax.jit
def kernel(cost, inputs, mask):
    return _nll_sc(cost.T, inputs.astype(jnp.int32), mask)


# speedup vs baseline: 42.2071x; 1.1227x over previous
import functools

import jax
import jax.numpy as jnp
from jax import lax
from jax.experimental import pallas as pl
from jax.experimental.pallas import tpu as pltpu
from jax.experimental.pallas import tpu_sc as plsc

B = 1024
V = 100000
L = 16
NC, NS = 2, 16
NW = NC * NS
BPW = B // NW
TR, TL = 8, 128


@functools.partial(
    pl.kernel,
    mesh=plsc.VectorSubcoreMesh(core_axis_name="c", subcore_axis_name="s"),
    out_type=jax.ShapeDtypeStruct((B,), jnp.float32),
    compiler_params=pltpu.CompilerParams(needs_layout_passes=False),
    scratch_types=[
        pltpu.VMEM((BPW,), jnp.int32),
        pltpu.VMEM((BPW,), jnp.int32),
        pltpu.VMEM((BPW,), jnp.float32),
        pltpu.VMEM((BPW,), jnp.float32),
        pltpu.VMEM((BPW,), jnp.float32),
        pltpu.SemaphoreType.DMA,
    ],
)
def _nll_sc(flat_hbm, inputs_hbm, mask_hbm, out_hbm,
            idx_v, phys_v, vals_v, mask_v, out_v, sem):
    wid = lax.axis_index("s") * NC + lax.axis_index("c")
    base = wid * BPW

    pltpu.sync_copy(inputs_hbm.at[pl.ds(base, BPW)], idx_v)
    pltpu.sync_copy(mask_hbm.at[pl.ds(base, BPW)], mask_v)

    for j in range(BPW // L):
        sl = pl.ds(j * L, L)
        i = base + j * L + lax.iota(jnp.int32, L)
        v = idx_v[sl]
        tile = lax.bitwise_and(v, ~(TR - 1)) + lax.shift_right_logical(i, 7)
        intra = lax.bitwise_and(v, TR - 1) * TL + lax.bitwise_and(i, TL - 1)
        phys_v[sl] = tile * (TR * TL) + intra

    pltpu.async_copy(flat_hbm.at[phys_v], vals_v, sem).wait()

    for j in range(BPW // L):
        sl = pl.ds(j * L, L)
        out_v[sl] = -vals_v[sl] * mask_v[sl]

    pltpu.sync_copy(out_v, out_hbm.at[pl.ds(base, BPW)])


@jax.jit
def kernel(cost, inputs, mask):
    flat = (cost.T.reshape(V // TR, TR, B // TL, TL)
            .transpose(0, 2, 1, 3).reshape(B * V))
    return _nll_sc(flat, inputs.astype(jnp.int32), mask)


# speedup vs baseline: 45.2469x; 1.0720x over previous
import functools

import jax
import jax.numpy as jnp
from jax import lax
from jax.experimental import pallas as pl
from jax.experimental.pallas import tpu as pltpu
from jax.experimental.pallas import tpu_sc as plsc

B = 1024
V = 100000
L = 16
NC, NS = 1, 16
NW = NC * NS
BPW = B // NW
TR, TL = 8, 128


@functools.partial(
    pl.kernel,
    mesh=plsc.VectorSubcoreMesh(
        core_axis_name="c", subcore_axis_name="s", num_cores=NC),
    out_type=jax.ShapeDtypeStruct((B,), jnp.float32),
    compiler_params=pltpu.CompilerParams(needs_layout_passes=False),
    scratch_types=[
        pltpu.VMEM((BPW,), jnp.int32),
        pltpu.VMEM((BPW,), jnp.int32),
        pltpu.VMEM((BPW,), jnp.float32),
        pltpu.VMEM((BPW,), jnp.float32),
        pltpu.VMEM((BPW,), jnp.float32),
        pltpu.SemaphoreType.DMA,
    ],
)
def _nll_sc(flat_hbm, inputs_hbm, mask_hbm, out_hbm,
            idx_v, phys_v, vals_v, mask_v, out_v, sem):
    wid = lax.axis_index("s") * NC + lax.axis_index("c")
    base = wid * BPW

    pltpu.sync_copy(inputs_hbm.at[pl.ds(base, BPW)], idx_v)
    pltpu.sync_copy(mask_hbm.at[pl.ds(base, BPW)], mask_v)

    for j in range(BPW // L):
        sl = pl.ds(j * L, L)
        i = base + j * L + lax.iota(jnp.int32, L)
        v = idx_v[sl]
        tile = lax.bitwise_and(v, ~(TR - 1)) + lax.shift_right_logical(i, 7)
        intra = lax.bitwise_and(v, TR - 1) * TL + lax.bitwise_and(i, TL - 1)
        phys_v[sl] = tile * (TR * TL) + intra

    pltpu.async_copy(flat_hbm.at[phys_v], vals_v, sem).wait()

    for j in range(BPW // L):
        sl = pl.ds(j * L, L)
        out_v[sl] = -vals_v[sl] * mask_v[sl]

    pltpu.sync_copy(out_v, out_hbm.at[pl.ds(base, BPW)])


@jax.jit
def kernel(cost, inputs, mask):
    flat = (cost.T.reshape(V // TR, TR, B // TL, TL)
            .transpose(0, 2, 1, 3).reshape(B * V))
    return _nll_sc(flat, inputs.astype(jnp.int32), mask)


# speedup vs baseline: 106.1657x; 2.3464x over previous
import functools

import jax
import jax.numpy as jnp
from jax import lax
from jax.experimental import pallas as pl
from jax.experimental.pallas import tpu as pltpu

B = 1024
V = 100000
TR, TL = 8, 128
NROW = B * V // TL
UNROLL = 8


def _body(flat2, inp_s, inp_v, mask_v, out_v, gathered, sem):
    def issue(j, carry):
        for u in range(UNROLL):
            jj = j * UNROLL + u
            v = inp_s[jj]
            tile = lax.bitwise_and(v, ~(TR - 1)) + lax.shift_right_logical(jj, 7)
            phys = (tile * (TR * TL)
                    + lax.bitwise_and(v, TR - 1) * TL
                    + lax.bitwise_and(jj, TL - 1))
            row = lax.shift_right_logical(phys, 7)
            pltpu.make_async_copy(
                flat2.at[pl.ds(row, 1), :],
                gathered.at[pl.ds(jj, 1), :],
                sem,
            ).start()
        return carry

    lax.fori_loop(0, B // UNROLL, issue, 0)

    def drain(j, carry):
        pltpu.make_async_copy(
            flat2.at[pl.ds(0, 1), :],
            gathered.at[pl.ds(0, 1), :],
            sem,
        ).wait()
        return carry

    lax.fori_loop(0, B, drain, 0, unroll=8)

    i_vec = lax.broadcasted_iota(jnp.int32, (B,), 0)
    v = inp_v[...]
    tile = lax.bitwise_and(v, ~(TR - 1)) + lax.shift_right_logical(i_vec, 7)
    phys = (tile * (TR * TL)
            + lax.bitwise_and(v, TR - 1) * TL
            + lax.bitwise_and(i_vec, TL - 1))
    lane = lax.bitwise_and(phys, TL - 1)
    sel = jnp.where(
        lane[:, None] == lax.broadcasted_iota(jnp.int32, (B, TL), 1),
        gathered[...], 0.0)
    vals = jnp.sum(sel, axis=1)
    out_v[...] = -vals * mask_v[...]


_gather_tc = pl.pallas_call(
    _body,
    out_shape=jax.ShapeDtypeStruct((B,), jnp.float32),
    in_specs=[
        pl.BlockSpec(memory_space=pl.ANY),
        pl.BlockSpec(memory_space=pltpu.SMEM),
        pl.BlockSpec(memory_space=pltpu.VMEM),
        pl.BlockSpec(memory_space=pltpu.VMEM),
    ],
    out_specs=pl.BlockSpec(memory_space=pltpu.VMEM),
    scratch_shapes=[
        pltpu.VMEM((B, TL), jnp.float32),
        pltpu.SemaphoreType.DMA,
    ],
)


@jax.jit
def kernel(cost, inputs, mask):
    flat2 = (cost.T.reshape(V // TR, TR, B // TL, TL)
             .transpose(0, 2, 1, 3).reshape(NROW, TL))
    inputs = inputs.astype(jnp.int32)
    return _gather_tc(flat2, inputs, inputs, mask)


# speedup vs baseline: 122.4532x; 1.1534x over previous
import jax
import jax.numpy as jnp
from jax import lax
from jax.experimental import pallas as pl
from jax.experimental.pallas import tpu as pltpu

B = 1024
V = 100000
TR, TL = 8, 128
NROW = B * V // TL
NBLK = B // TL
UNROLL = 8


def _body(flat2, inp_s, inp_v, mask_v, out_v, gathered, sem):
    for b in range(NBLK):
        def issue(j, carry, b=b):
            for u in range(UNROLL):
                jj = b * TL + j * UNROLL + u
                v = inp_s[jj]
                row = (lax.shift_left(lax.bitwise_and(v, ~(TR - 1)), 3)
                       + lax.bitwise_and(v, TR - 1) + b * TR)
                pltpu.make_async_copy(
                    flat2.at[pl.ds(row, 1), :],
                    gathered.at[pl.ds(jj, 1), :],
                    sem,
                ).start()
            return carry

        lax.fori_loop(0, TL // UNROLL, issue, 0)

    for _ in range(NBLK):
        pltpu.make_async_copy(
            flat2.at[pl.ds(0, TL), :],
            gathered.at[pl.ds(0, TL), :],
            sem,
        ).wait()

    i_vec = lax.broadcasted_iota(jnp.int32, (B,), 0)
    lane = lax.bitwise_and(i_vec, TL - 1)
    sel = jnp.where(
        lane[:, None] == lax.broadcasted_iota(jnp.int32, (B, TL), 1),
        gathered[...], 0.0)
    vals = jnp.sum(sel, axis=1)
    out_v[...] = -vals * mask_v[...]


_gather_tc = pl.pallas_call(
    _body,
    out_shape=jax.ShapeDtypeStruct((B,), jnp.float32),
    in_specs=[
        pl.BlockSpec(memory_space=pl.ANY),
        pl.BlockSpec(memory_space=pltpu.SMEM),
        pl.BlockSpec(memory_space=pltpu.VMEM),
        pl.BlockSpec(memory_space=pltpu.VMEM),
    ],
    out_specs=pl.BlockSpec(memory_space=pltpu.VMEM),
    scratch_shapes=[
        pltpu.VMEM((B, TL), jnp.float32),
        pltpu.SemaphoreType.DMA,
    ],
)


@jax.jit
def kernel(cost, inputs, mask):
    flat2 = (cost.T.reshape(V // TR, TR, B // TL, TL)
             .transpose(0, 2, 1, 3).reshape(NROW, TL))
    inputs = inputs.astype(jnp.int32)
    return _gather_tc(flat2, inputs, inputs, mask)
